# baseline (speedup 1.0000x reference)
import jax
import jax.numpy as jnp
from jax.experimental import pallas as pl

N_EMBED = 8192
EMBED_DIM = 32
DEPTH = 8
TOK_BLOCK = 384
_CHUNKS = 128
_CHUNK_ROWS = N_EMBED // _CHUNKS
_CHUNK_W = _CHUNK_ROWS * EMBED_DIM


_NCHAIN = 1


def _rvq_block(tok_ref, cbt_ref, cbr_ref, cn_ref, out_ref, codes_ref):
    T = TOK_BLOCK // _NCHAIN
    iota_hi = jax.lax.broadcasted_iota(jnp.int32, (T, _CHUNKS), 1)
    group = jax.lax.broadcasted_iota(jnp.int32, (T, _CHUNK_W), 1) // EMBED_DIM
    for h in range(_NCHAIN):
        sl = pl.ds(h * T, T)
        resid = tok_ref[sl, :]
        x0 = resid
        agg = jnp.zeros_like(resid)
        for i in range(DEPTH):
            cn = cn_ref[i][None, :]
            xn = jnp.sum(resid * resid, axis=1, keepdims=True)
            dot2 = jax.lax.dot_general(
                resid, cbt_ref[i], (((1,), (0,)), ((), ())),
                preferred_element_type=jnp.float32,
                precision=jax.lax.Precision.DEFAULT)
            dist = (xn + cn) + dot2
            idx = jnp.argmin(dist, axis=1)
            hi = idx // _CHUNK_ROWS
            lo = idx % _CHUNK_ROWS
            oh_hi = (iota_hi == hi[:, None]).astype(jnp.float32)
            chunk = jax.lax.dot_general(
                oh_hi, cbr_ref[i], (((1,), (0,)), ((), ())),
                preferred_element_type=jnp.float32,
                precision=jax.lax.Precision.HIGHEST)
            picked = jnp.where(group == lo[:, None], chunk, 0.0)
            w = _CHUNK_W
            while w > EMBED_DIM:
                w //= 2
                picked = picked[:, :w] + picked[:, w:2 * w]
            quant = picked
            resid = resid - quant
            agg = agg + quant
            codes_ref[i, sl] = idx
        out_ref[sl, :] = x0 + (agg - x0)


def kernel(x, codebooks):
    B, C, H, W = x.shape
    n_tok = B * H * W
    tokens = jnp.transpose(x, (0, 2, 3, 1)).reshape(n_tok, C)
    cn = jnp.sum(codebooks * codebooks, axis=2)
    cbt = -2.0 * jnp.transpose(codebooks, (0, 2, 1))
    cbr = codebooks.reshape(DEPTH, _CHUNKS, _CHUNK_W)
    grid = (n_tok // TOK_BLOCK,)
    out_tok, codes = pl.pallas_call(
        _rvq_block,
        grid=grid,
        in_specs=[
            pl.BlockSpec((TOK_BLOCK, C), lambda i: (i, 0)),
            pl.BlockSpec((DEPTH, EMBED_DIM, N_EMBED), lambda i: (0, 0, 0)),
            pl.BlockSpec((DEPTH, _CHUNKS, _CHUNK_W), lambda i: (0, 0, 0)),
            pl.BlockSpec((DEPTH, N_EMBED), lambda i: (0, 0)),
        ],
        out_specs=[
            pl.BlockSpec((TOK_BLOCK, C), lambda i: (i, 0)),
            pl.BlockSpec((DEPTH, TOK_BLOCK), lambda i: (0, i)),
        ],
        out_shape=[
            jax.ShapeDtypeStruct((n_tok, C), jnp.float32),
            jax.ShapeDtypeStruct((DEPTH, n_tok), jnp.int32),
        ],
    )(tokens, cbt, cbr, cn)
    out = out_tok.reshape(B, H, W, C).transpose(0, 3, 1, 2)
    codes = codes.T.reshape(B, H, W, DEPTH).astype(jnp.int64)
    return out, codes


# speedup vs baseline: 1.1794x; 1.1794x over previous
import jax
import jax.numpy as jnp
from jax.experimental import pallas as pl

N_EMBED = 8192
EMBED_DIM = 32
DEPTH = 8
TOK_BLOCK = 256
_CHUNKS = 128
_CHUNK_ROWS = N_EMBED // _CHUNKS
_CHUNK_W = _CHUNK_ROWS * EMBED_DIM


_NCHAIN = 1


def _rvq_block(tok_ref, cbt_ref, cbr_ref, cn_ref, out_ref, codes_ref):
    T = TOK_BLOCK // _NCHAIN
    iota_hi = jax.lax.broadcasted_iota(jnp.int32, (T, _CHUNKS), 1)
    group = jax.lax.broadcasted_iota(jnp.int32, (T, _CHUNK_W), 1) // EMBED_DIM
    for h in range(_NCHAIN):
        sl = pl.ds(h * T, T)
        resid = tok_ref[sl, :]
        x0 = resid
        agg = jnp.zeros_like(resid)
        for i in range(DEPTH):
            cn = cn_ref[i][None, :]
            xn = jnp.sum(resid * resid, axis=1, keepdims=True)
            dot2 = jax.lax.dot_general(
                resid, cbt_ref[i], (((1,), (0,)), ((), ())),
                preferred_element_type=jnp.float32,
                precision=jax.lax.Precision.DEFAULT)
            dist = (xn + cn) + dot2
            idx = jnp.argmin(dist, axis=1)
            hi = idx // _CHUNK_ROWS
            lo = idx % _CHUNK_ROWS
            oh_hi = (iota_hi == hi[:, None]).astype(jnp.float32)
            chunk = jax.lax.dot_general(
                oh_hi, cbr_ref[i], (((1,), (0,)), ((), ())),
                preferred_element_type=jnp.float32,
                precision=jax.lax.Precision.HIGHEST)
            picked = jnp.where(group == lo[:, None], chunk, 0.0)
            w = _CHUNK_W
            while w > EMBED_DIM:
                w //= 2
                picked = picked[:, :w] + picked[:, w:2 * w]
            quant = picked
            resid = resid - quant
            agg = agg + quant
            codes_ref[i, sl] = idx
        out_ref[sl, :] = x0 + (agg - x0)


def kernel(x, codebooks):
    B, C, H, W = x.shape
    n_tok = B * H * W
    tokens = jnp.transpose(x, (0, 2, 3, 1)).reshape(n_tok, C)
    cn = jnp.sum(codebooks * codebooks, axis=2)
    cbt = -2.0 * jnp.transpose(codebooks, (0, 2, 1))
    cbr = codebooks.reshape(DEPTH, _CHUNKS, _CHUNK_W)
    grid = (n_tok // TOK_BLOCK,)
    out_tok, codes = pl.pallas_call(
        _rvq_block,
        grid=grid,
        in_specs=[
            pl.BlockSpec((TOK_BLOCK, C), lambda i: (i, 0)),
            pl.BlockSpec((DEPTH, EMBED_DIM, N_EMBED), lambda i: (0, 0, 0)),
            pl.BlockSpec((DEPTH, _CHUNKS, _CHUNK_W), lambda i: (0, 0, 0)),
            pl.BlockSpec((DEPTH, N_EMBED), lambda i: (0, 0)),
        ],
        out_specs=[
            pl.BlockSpec((TOK_BLOCK, C), lambda i: (i, 0)),
            pl.BlockSpec((DEPTH, TOK_BLOCK), lambda i: (0, i)),
        ],
        out_shape=[
            jax.ShapeDtypeStruct((n_tok, C), jnp.float32),
            jax.ShapeDtypeStruct((DEPTH, n_tok), jnp.int32),
        ],
    )(tokens, cbt, cbr, cn)
    out = out_tok.reshape(B, H, W, C).transpose(0, 3, 1, 2)
    codes = codes.T.reshape(B, H, W, DEPTH).astype(jnp.int64)
    return out, codes


# speedup vs baseline: 1.4676x; 1.2444x over previous
import jax
import jax.numpy as jnp
from jax.experimental import pallas as pl

N_EMBED = 8192
EMBED_DIM = 32
DEPTH = 8
TOK_BLOCK = 256
_CHUNKS = 256
_CHUNK_ROWS = N_EMBED // _CHUNKS
_CHUNK_W = _CHUNK_ROWS * EMBED_DIM


_NCHAIN = 1


def _rvq_block(tok_ref, cbt_ref, cbr_ref, cn_ref, out_ref, codes_ref):
    T = TOK_BLOCK // _NCHAIN
    iota_hi = jax.lax.broadcasted_iota(jnp.int32, (T, _CHUNKS), 1)
    group = jax.lax.broadcasted_iota(jnp.int32, (T, _CHUNK_W), 1) // EMBED_DIM
    for h in range(_NCHAIN):
        sl = pl.ds(h * T, T)
        resid = tok_ref[sl, :]
        x0 = resid
        agg = jnp.zeros_like(resid)
        for i in range(DEPTH):
            cn = cn_ref[i][None, :]
            xn = jnp.sum(resid * resid, axis=1, keepdims=True)
            dot2 = jax.lax.dot_general(
                resid, cbt_ref[i], (((1,), (0,)), ((), ())),
                preferred_element_type=jnp.float32,
                precision=jax.lax.Precision.DEFAULT)
            dist = (xn + cn) + dot2
            idx = jnp.argmin(dist, axis=1)
            hi = idx // _CHUNK_ROWS
            lo = idx % _CHUNK_ROWS
            oh_hi = (iota_hi == hi[:, None]).astype(jnp.float32)
            chunk = jax.lax.dot_general(
                oh_hi, cbr_ref[i], (((1,), (0,)), ((), ())),
                preferred_element_type=jnp.float32,
                precision=jax.lax.Precision.HIGHEST)
            picked = jnp.where(group == lo[:, None], chunk, 0.0)
            w = _CHUNK_W
            while w > EMBED_DIM:
                w //= 2
                picked = picked[:, :w] + picked[:, w:2 * w]
            quant = picked
            resid = resid - quant
            agg = agg + quant
            codes_ref[i, sl] = idx
        out_ref[sl, :] = x0 + (agg - x0)


def kernel(x, codebooks):
    B, C, H, W = x.shape
    n_tok = B * H * W
    tokens = jnp.transpose(x, (0, 2, 3, 1)).reshape(n_tok, C)
    cn = jnp.sum(codebooks * codebooks, axis=2)
    cbt = -2.0 * jnp.transpose(codebooks, (0, 2, 1))
    cbr = codebooks.reshape(DEPTH, _CHUNKS, _CHUNK_W)
    grid = (n_tok // TOK_BLOCK,)
    out_tok, codes = pl.pallas_call(
        _rvq_block,
        grid=grid,
        in_specs=[
            pl.BlockSpec((TOK_BLOCK, C), lambda i: (i, 0)),
            pl.BlockSpec((DEPTH, EMBED_DIM, N_EMBED), lambda i: (0, 0, 0)),
            pl.BlockSpec((DEPTH, _CHUNKS, _CHUNK_W), lambda i: (0, 0, 0)),
            pl.BlockSpec((DEPTH, N_EMBED), lambda i: (0, 0)),
        ],
        out_specs=[
            pl.BlockSpec((TOK_BLOCK, C), lambda i: (i, 0)),
            pl.BlockSpec((DEPTH, TOK_BLOCK), lambda i: (0, i)),
        ],
        out_shape=[
            jax.ShapeDtypeStruct((n_tok, C), jnp.float32),
            jax.ShapeDtypeStruct((DEPTH, n_tok), jnp.int32),
        ],
    )(tokens, cbt, cbr, cn)
    out = out_tok.reshape(B, H, W, C).transpose(0, 3, 1, 2)
    codes = codes.T.reshape(B, H, W, DEPTH).astype(jnp.int64)
    return out, codes
